# Initial kernel scaffold; baseline (speedup 1.0000x reference)
import jax
import jax.numpy as jnp
from jax.experimental import pallas as pl


def kernel(qk, pos_bias, pos_indices):
    raise NotImplementedError("write your pallas kernel here")


# speedup vs baseline: 15.6972x; 15.6972x over previous
import functools

import jax
import jax.numpy as jnp
from jax import lax
from jax.experimental import pallas as pl
from jax.experimental.pallas import tpu as pltpu
from jax.experimental.pallas import tpu_sc as plsc

_H = 16
_N = 1024 * 1024
_NC, _NS, _L = 2, 16, 16
_NW = _NC * _NS
_PW = _N // _NW
_BLK = 2048
_TW = 3976


@functools.partial(
    pl.kernel,
    mesh=plsc.VectorSubcoreMesh(core_axis_name="c", subcore_axis_name="s"),
    out_type=jax.ShapeDtypeStruct((_H, _N), jnp.float32),
    scratch_types=[
        pltpu.VMEM((_H * _TW,), jnp.float32),
        pltpu.VMEM((_BLK,), jnp.int32),
        pltpu.VMEM((_H, _BLK), jnp.float32),
    ],
    compiler_params=pltpu.CompilerParams(needs_layout_passes=False),
)
def _sc_lookup(tab_hbm, idx_hbm, out_hbm, tab_v, idx_v, out_v):
    wid = lax.axis_index("s") * _NC + lax.axis_index("c")
    base = wid * _PW
    pltpu.sync_copy(tab_hbm, tab_v)
    for blk in range(_PW // _BLK):
        pltpu.sync_copy(idx_hbm.at[pl.ds(base + blk * _BLK, _BLK)], idx_v)

        def body(g, carry):
            iv = idx_v[pl.ds(g * _L, _L)]
            for h in range(_H):
                val = plsc.load_gather(tab_v, [iv + h * _TW])
                out_v[h, pl.ds(g * _L, _L)] = val
            return carry

        lax.fori_loop(0, _BLK // _L, body, 0)
        pltpu.sync_copy(out_v, out_hbm.at[:, pl.ds(base + blk * _BLK, _BLK)])


def kernel(qk, pos_bias, pos_indices):
    del qk
    tab = jnp.pad(jnp.transpose(pos_bias), ((0, 0), (0, _TW - pos_bias.shape[0])))
    out = _sc_lookup(tab.reshape(-1), pos_indices.reshape(-1))
    return out.reshape(_H, 1024, 1024)
